# Initial kernel scaffold; baseline (speedup 1.0000x reference)
import jax
import jax.numpy as jnp
from jax.experimental import pallas as pl


def kernel(feats, edge_index, s):
    raise NotImplementedError("write your pallas kernel here")


# speedup vs baseline: 2.4143x; 2.4143x over previous
import functools

import jax
import jax.numpy as jnp
from jax import lax
from jax.experimental import pallas as pl
from jax.experimental.pallas import tpu as pltpu
from jax.experimental.pallas import tpu_sc as plsc

_N, _D, _K = 10000, 128, 10
_NSUB = 16
_NCORE = 2
_NW = _NSUB * _NCORE
_CHUNK = 128
_NPAD = 10240
_RPW = _NPAD // _NW
_PPR = 80
_PPC = _RPW // _PPR
_JUNK = _RPW


def _rsqrt_iter(x):
    t = 1.0 / x
    y = x * 0.0 + 1.0
    for _ in range(16):
        y = 0.5 * (y + t / y)
    return y


def _make_sc_prop(ech):
    mesh = plsc.VectorSubcoreMesh(core_axis_name="c", subcore_axis_name="s",
                                  num_cores=_NCORE, num_subcores=_NSUB)

    @functools.partial(
        pl.kernel,
        out_type=(
            jax.ShapeDtypeStruct(((_K + 1) * _NPAD, _D), jnp.float32),
            jax.ShapeDtypeStruct((_NPAD, _D), jnp.float32),
        ),
        mesh=mesh,
        compiler_params=pltpu.CompilerParams(use_tc_tiling_on_sc=False),
        scratch_types=[
            pltpu.VMEM((_RPW + 8, _D), jnp.float32),
            pltpu.VMEM((_CHUNK, _D), jnp.float32),
            pltpu.VMEM((_PPR, _D), jnp.float32),
            pltpu.VMEM((_PPR, _D), jnp.float32),
            pltpu.VMEM((_RPW + 8, 16), jnp.float32),
            pltpu.VMEM((_RPW + 8, 16), jnp.float32),
            pltpu.VMEM((_CHUNK,), jnp.int32),
            pltpu.VMEM((_CHUNK,), jnp.int32),
            pltpu.VMEM((16,), jnp.int32),
            pltpu.SemaphoreType.DMA,
        ],
    )
    def sc_prop(feats_h, srcs_h, dsts_h, srcv_h, dlo_h, dhi_h, slo_h, shi_h,
                zeros_h,
                H_h, g_h,
                slab, gbuf, h_b, g_b, dnorm, snorm, idx_st, own_st, bnd_st,
                sem):
        c = lax.axis_index("c")
        t = lax.axis_index("s")
        w = c * _NSUB + t
        n0 = w * _RPW

        def bound(tbl):
            pltpu.sync_copy(tbl.at[w], bnd_st)
            return bnd_st[pl.ds(0, 16)][0]

        d_lo = bound(dlo_h)
        d_hi = bound(dhi_h)
        s_lo = bound(slo_h)
        s_hi = bound(shi_h)

        def count(vals_h, lo, hi, norm_v):
            def zrow(r, carry):
                norm_v[r, pl.ds(0, 16)] = jnp.zeros((16,), jnp.float32)
                return carry
            lax.fori_loop(0, _RPW + 8, zrow, 0)
            one16 = jnp.full((16,), 1.0, jnp.float32)

            def cbody(j, carry):
                pltpu.sync_copy(vals_h.at[j], own_st)

                def ev_body(ev, ecarry):
                    vv = own_st[pl.ds(ev * 16, 16)]
                    for lane in range(16):
                        il = vv[lane] - n0
                        ok = jnp.logical_and(il >= 0, il < _RPW)
                        ils = jnp.where(ok, il, _JUNK)
                        plsc.addupdate(norm_v.at[ils, pl.ds(0, 16)], one16)
                    return ecarry
                lax.fori_loop(0, _CHUNK // 16, ev_body, 0)
                return carry
            lax.fori_loop(lo, hi, cbody, 0)

            def nrow(r, carry):
                x = jnp.maximum(norm_v[r, pl.ds(0, 16)], 1.0)
                norm_v[r, pl.ds(0, 16)] = _rsqrt_iter(x)
                return carry
            lax.fori_loop(0, _RPW, nrow, 0)
        count(dsts_h, d_lo, d_hi, dnorm)
        count(srcv_h, s_lo, s_hi, snorm)

        def hop0_chunk(q, carry):
            base = n0 + q * _PPR
            pltpu.sync_copy(zeros_h.at[pl.ds(q * _PPR, _PPR)],
                            slab.at[pl.ds(q * _PPR, _PPR)])
            pltpu.sync_copy(feats_h.at[pl.ds(base, _PPR)], h_b)
            pltpu.sync_copy(h_b, H_h.at[pl.ds(base, _PPR)])

            def g0_row(r, rcarry):
                sn = snorm[q * _PPR + r, pl.ds(0, 16)]
                for f in range(_D // 16):
                    g_b[r, pl.ds(f * 16, 16)] = h_b[r, pl.ds(f * 16, 16)] * sn
                return rcarry
            lax.fori_loop(0, _PPR, g0_row, 0)
            pltpu.sync_copy(g_b, g_h.at[pl.ds(base, _PPR)])
            return carry
        lax.fori_loop(0, _PPC, hop0_chunk, 0)
        plsc.subcore_barrier()

        def hop(k, carry):
            def chunk_body(j, ccarry):
                pltpu.sync_copy(srcs_h.at[j], idx_st)
                pltpu.sync_copy(dsts_h.at[j], own_st)
                pltpu.async_copy(g_h.at[idx_st], gbuf, sem).wait()

                def ev_body(ev, ecarry):
                    vv = own_st[pl.ds(ev * 16, 16)]
                    for lane in range(16):
                        il = vv[lane] - n0
                        ok = jnp.logical_and(il >= 0, il < _RPW)
                        ils = jnp.where(ok, il, _JUNK)
                        e = ev * 16 + lane
                        for f in range(_D // 16):
                            plsc.addupdate(slab.at[ils, pl.ds(f * 16, 16)],
                                           gbuf[e, pl.ds(f * 16, 16)])
                    return ecarry
                lax.fori_loop(0, _CHUNK // 16, ev_body, 0)
                return ccarry
            lax.fori_loop(d_lo, d_hi, chunk_body, 0)
            plsc.subcore_barrier()

            def pp_chunk(q, pcarry):
                base = n0 + q * _PPR

                def pp_row(r, rcarry):
                    dn = dnorm[q * _PPR + r, pl.ds(0, 16)]
                    sn = snorm[q * _PPR + r, pl.ds(0, 16)]
                    for f in range(_D // 16):
                        v = slab[q * _PPR + r, pl.ds(f * 16, 16)] * dn
                        h_b[r, pl.ds(f * 16, 16)] = v
                        g_b[r, pl.ds(f * 16, 16)] = v * sn
                    return rcarry
                lax.fori_loop(0, _PPR, pp_row, 0)
                hoff = (k + 1) * _NPAD + base
                pltpu.sync_copy(h_b, H_h.at[pl.ds(hoff, _PPR)])
                pltpu.sync_copy(g_b, g_h.at[pl.ds(base, _PPR)])
                pltpu.sync_copy(zeros_h.at[pl.ds(q * _PPR, _PPR)],
                                slab.at[pl.ds(q * _PPR, _PPR)])
                return pcarry
            lax.fori_loop(0, _PPC, pp_chunk, 0)
            plsc.subcore_barrier()
            return carry
        lax.fori_loop(0, _K, hop, 0)

    return sc_prop


_BLK = 2000


def _pool_body(H_ref, s_ref, o_ref):
    Hb = H_ref[...]
    s = s_ref[...]
    dn = (((2,), (0,)), ((), ()))
    logits = lax.dot_general(Hb, s[:, 0], dn,
                             preferred_element_type=jnp.float32)
    sig = jax.nn.sigmoid(logits)
    o_ref[...] = jnp.sum(sig[:, :, None] * Hb, axis=0)


def _pool(H3, s):
    return pl.pallas_call(
        _pool_body,
        grid=(_N // _BLK,),
        in_specs=[
            pl.BlockSpec((_K + 1, _BLK, _D), lambda i: (0, i, 0)),
            pl.BlockSpec((_D, 1), lambda i: (0, 0)),
        ],
        out_specs=pl.BlockSpec((_BLK, _D), lambda i: (i, 0)),
        out_shape=jax.ShapeDtypeStruct((_N, _D), jnp.float32),
    )(H3, s)


def kernel(feats, edge_index, s):
    E = edge_index.shape[1]
    ech = -(-E // _CHUNK)
    epad = ech * _CHUNK

    src = edge_index[0]
    dst = edge_index[1]
    pad = jnp.full((epad - E,), _N, jnp.int32)
    src_p = jnp.concatenate([src, pad])
    dst_p = jnp.concatenate([dst, pad])

    order_d = jnp.argsort(dst_p)
    srcs = src_p[order_d].reshape(ech, _CHUNK)
    dsts_f = dst_p[order_d]
    dsts = dsts_f.reshape(ech, _CHUNK)
    srcv_f = jnp.sort(src_p)
    srcv = srcv_f.reshape(ech, _CHUNK)

    rng = jnp.arange(_NW + 1, dtype=jnp.int32) * _RPW
    d_bnd = jnp.searchsorted(dsts_f, rng).astype(jnp.int32)
    s_bnd = jnp.searchsorted(srcv_f, rng).astype(jnp.int32)

    def bcast(x):
        return jnp.broadcast_to(x[:, None], (_NW, 16)).astype(jnp.int32)

    dlo = bcast(d_bnd[:_NW] // _CHUNK)
    dhi = bcast(-(-d_bnd[1:] // _CHUNK))
    slo = bcast(s_bnd[:_NW] // _CHUNK)
    shi = bcast(-(-s_bnd[1:] // _CHUNK))

    feats_pad = jnp.zeros((_NPAD, _D), jnp.float32).at[:_N].set(feats)
    zeros_h = jnp.zeros((_RPW, _D), jnp.float32)

    H_flat, _ = _make_sc_prop(ech)(feats_pad, srcs, dsts, srcv,
                                   dlo, dhi, slo, shi, zeros_h)
    H3 = H_flat.reshape(_K + 1, _NPAD, _D)
    return _pool(H3, s)


# speedup vs baseline: 2.9169x; 1.2082x over previous
import functools

import jax
import jax.numpy as jnp
from jax import lax
from jax.experimental import pallas as pl
from jax.experimental.pallas import tpu as pltpu
from jax.experimental.pallas import tpu_sc as plsc

_N, _D, _K = 10000, 128, 10
_NSUB = 16
_NCORE = 2
_NW = _NSUB * _NCORE
_CHUNK = 128
_NPAD = 10240
_RPW = _NPAD // _NW
_PPR = 80
_PPC = _RPW // _PPR
_JUNK = _RPW


def _rsqrt_iter(x):
    t = 1.0 / x
    y = x * 0.0 + 1.0
    for _ in range(16):
        y = 0.5 * (y + t / y)
    return y


def _make_sc_prop(ech):
    mesh = plsc.VectorSubcoreMesh(core_axis_name="c", subcore_axis_name="s",
                                  num_cores=_NCORE, num_subcores=_NSUB)

    @functools.partial(
        pl.kernel,
        out_type=(
            jax.ShapeDtypeStruct(((_K + 1) * _NPAD, _D), jnp.float32),
            jax.ShapeDtypeStruct((_NPAD, _D), jnp.float32),
        ),
        mesh=mesh,
        compiler_params=pltpu.CompilerParams(use_tc_tiling_on_sc=False),
        scratch_types=[
            pltpu.VMEM((_RPW + 8, _D), jnp.float32),
            pltpu.VMEM((_CHUNK, _D), jnp.float32),
            pltpu.VMEM((_CHUNK, _D), jnp.float32),
            pltpu.VMEM((_PPR, _D), jnp.float32),
            pltpu.VMEM((_PPR, _D), jnp.float32),
            pltpu.VMEM((_RPW + 8, 16), jnp.float32),
            pltpu.VMEM((_RPW + 8, 16), jnp.float32),
            pltpu.VMEM((_CHUNK,), jnp.int32),
            pltpu.VMEM((_CHUNK,), jnp.int32),
            pltpu.VMEM((_CHUNK,), jnp.int32),
            pltpu.VMEM((_CHUNK,), jnp.int32),
            pltpu.VMEM((16,), jnp.int32),
            pltpu.SemaphoreType.DMA,
            pltpu.SemaphoreType.DMA,
        ],
    )
    def sc_prop(feats_h, srcs_h, dsts_h, srcv_h, dlo_h, dhi_h, slo_h, shi_h,
                zeros_h,
                H_h, g_h,
                slab, gbuf0, gbuf1, h_b, g_b, dnorm, snorm,
                idx0, idx1, own0, own1, bnd_st,
                sem0, sem1):
        c = lax.axis_index("c")
        t = lax.axis_index("s")
        w = c * _NSUB + t
        n0 = w * _RPW

        def bound(tbl):
            pltpu.sync_copy(tbl.at[w], bnd_st)
            return bnd_st[pl.ds(0, 16)][0]

        d_lo = bound(dlo_h)
        d_hi = bound(dhi_h)
        s_lo = bound(slo_h)
        s_hi = bound(shi_h)

        own_st = own0

        def count(vals_h, lo, hi, norm_v):
            def zrow(r, carry):
                norm_v[r, pl.ds(0, 16)] = jnp.zeros((16,), jnp.float32)
                return carry
            lax.fori_loop(0, _RPW + 8, zrow, 0)
            one16 = jnp.full((16,), 1.0, jnp.float32)

            def cbody(j, carry):
                pltpu.sync_copy(vals_h.at[j], own_st)

                def ev_body(ev, ecarry):
                    vv = own_st[pl.ds(ev * 16, 16)]
                    for lane in range(16):
                        il = vv[lane] - n0
                        ok = jnp.logical_and(il >= 0, il < _RPW)
                        ils = jnp.where(ok, il, _JUNK)
                        plsc.addupdate(norm_v.at[ils, pl.ds(0, 16)], one16)
                    return ecarry
                lax.fori_loop(0, _CHUNK // 16, ev_body, 0)
                return carry
            lax.fori_loop(lo, hi, cbody, 0)

            def nrow(r, carry):
                x = jnp.maximum(norm_v[r, pl.ds(0, 16)], 1.0)
                norm_v[r, pl.ds(0, 16)] = _rsqrt_iter(x)
                return carry
            lax.fori_loop(0, _RPW, nrow, 0)
        count(dsts_h, d_lo, d_hi, dnorm)
        count(srcv_h, s_lo, s_hi, snorm)

        def hop0_chunk(q, carry):
            base = n0 + q * _PPR
            pltpu.sync_copy(zeros_h.at[pl.ds(q * _PPR, _PPR)],
                            slab.at[pl.ds(q * _PPR, _PPR)])
            pltpu.sync_copy(feats_h.at[pl.ds(base, _PPR)], h_b)
            pltpu.sync_copy(h_b, H_h.at[pl.ds(base, _PPR)])

            def g0_row(r, rcarry):
                sn = snorm[q * _PPR + r, pl.ds(0, 16)]
                for f in range(_D // 16):
                    g_b[r, pl.ds(f * 16, 16)] = h_b[r, pl.ds(f * 16, 16)] * sn
                return rcarry
            lax.fori_loop(0, _PPR, g0_row, 0)
            pltpu.sync_copy(g_b, g_h.at[pl.ds(base, _PPR)])
            return carry
        lax.fori_loop(0, _PPC, hop0_chunk, 0)
        plsc.subcore_barrier()

        def start(j, idx, own, gbuf, sem):
            pltpu.sync_copy(srcs_h.at[j], idx)
            pltpu.sync_copy(dsts_h.at[j], own)
            pltpu.async_copy(g_h.at[idx], gbuf, sem)

        def drain(idx, gbuf, sem):
            pltpu.make_async_copy(g_h.at[idx], gbuf, sem).wait()

        def accum(own, gbuf):
            def ev_body(ev, ecarry):
                vv = own[pl.ds(ev * 16, 16)]
                for lane in range(16):
                    il = vv[lane] - n0
                    ok = jnp.logical_and(il >= 0, il < _RPW)
                    ils = jnp.where(ok, il, _JUNK)
                    e = ev * 16 + lane
                    for f in range(_D // 16):
                        plsc.addupdate(slab.at[ils, pl.ds(f * 16, 16)],
                                       gbuf[e, pl.ds(f * 16, 16)])
                return ecarry
            lax.fori_loop(0, _CHUNK // 16, ev_body, 0)

        def hop(k, carry):
            @pl.when(d_lo < d_hi)
            def _():
                start(d_lo, idx0, own0, gbuf0, sem0)

            def pair_body(p, pcarry):
                j0 = d_lo + 2 * p
                j1 = j0 + 1

                @pl.when(j1 < d_hi)
                def _():
                    start(j1, idx1, own1, gbuf1, sem1)
                drain(idx0, gbuf0, sem0)
                accum(own0, gbuf0)

                @pl.when(j0 + 2 < d_hi)
                def _():
                    start(j0 + 2, idx0, own0, gbuf0, sem0)

                @pl.when(j1 < d_hi)
                def _():
                    drain(idx1, gbuf1, sem1)
                    accum(own1, gbuf1)
                return pcarry
            lax.fori_loop(0, (d_hi - d_lo + 1) // 2, pair_body, 0)
            plsc.subcore_barrier()

            def pp_chunk(q, pcarry):
                base = n0 + q * _PPR

                def pp_row(r, rcarry):
                    dn = dnorm[q * _PPR + r, pl.ds(0, 16)]
                    sn = snorm[q * _PPR + r, pl.ds(0, 16)]
                    for f in range(_D // 16):
                        v = slab[q * _PPR + r, pl.ds(f * 16, 16)] * dn
                        h_b[r, pl.ds(f * 16, 16)] = v
                        g_b[r, pl.ds(f * 16, 16)] = v * sn
                    return rcarry
                lax.fori_loop(0, _PPR, pp_row, 0)
                hoff = (k + 1) * _NPAD + base
                pltpu.sync_copy(h_b, H_h.at[pl.ds(hoff, _PPR)])
                pltpu.sync_copy(g_b, g_h.at[pl.ds(base, _PPR)])
                pltpu.sync_copy(zeros_h.at[pl.ds(q * _PPR, _PPR)],
                                slab.at[pl.ds(q * _PPR, _PPR)])
                return pcarry
            lax.fori_loop(0, _PPC, pp_chunk, 0)
            plsc.subcore_barrier()
            return carry
        lax.fori_loop(0, _K, hop, 0)

    return sc_prop


_BLK = 2000


def _pool_body(H_ref, s_ref, o_ref):
    Hb = H_ref[...]
    s = s_ref[...]
    dn = (((2,), (0,)), ((), ()))
    logits = lax.dot_general(Hb, s[:, 0], dn,
                             preferred_element_type=jnp.float32)
    sig = jax.nn.sigmoid(logits)
    o_ref[...] = jnp.sum(sig[:, :, None] * Hb, axis=0)


def _pool(H3, s):
    return pl.pallas_call(
        _pool_body,
        grid=(_N // _BLK,),
        in_specs=[
            pl.BlockSpec((_K + 1, _BLK, _D), lambda i: (0, i, 0)),
            pl.BlockSpec((_D, 1), lambda i: (0, 0)),
        ],
        out_specs=pl.BlockSpec((_BLK, _D), lambda i: (i, 0)),
        out_shape=jax.ShapeDtypeStruct((_N, _D), jnp.float32),
    )(H3, s)


def kernel(feats, edge_index, s):
    E = edge_index.shape[1]
    ech = -(-E // _CHUNK)
    epad = ech * _CHUNK

    src = edge_index[0]
    dst = edge_index[1]
    pad = jnp.full((epad - E,), _N, jnp.int32)
    src_p = jnp.concatenate([src, pad])
    dst_p = jnp.concatenate([dst, pad])

    order_d = jnp.argsort(dst_p)
    srcs = src_p[order_d].reshape(ech, _CHUNK)
    dsts_f = dst_p[order_d]
    dsts = dsts_f.reshape(ech, _CHUNK)
    srcv_f = jnp.sort(src_p)
    srcv = srcv_f.reshape(ech, _CHUNK)

    rng = jnp.arange(_NW + 1, dtype=jnp.int32) * _RPW
    d_bnd = jnp.searchsorted(dsts_f, rng).astype(jnp.int32)
    s_bnd = jnp.searchsorted(srcv_f, rng).astype(jnp.int32)

    def bcast(x):
        return jnp.broadcast_to(x[:, None], (_NW, 16)).astype(jnp.int32)

    dlo = bcast(d_bnd[:_NW] // _CHUNK)
    dhi = bcast(-(-d_bnd[1:] // _CHUNK))
    slo = bcast(s_bnd[:_NW] // _CHUNK)
    shi = bcast(-(-s_bnd[1:] // _CHUNK))

    feats_pad = jnp.zeros((_NPAD, _D), jnp.float32).at[:_N].set(feats)
    zeros_h = jnp.zeros((_RPW, _D), jnp.float32)

    H_flat, _ = _make_sc_prop(ech)(feats_pad, srcs, dsts, srcv,
                                   dlo, dhi, slo, shi, zeros_h)
    H3 = H_flat.reshape(_K + 1, _NPAD, _D)
    return _pool(H3, s)
